# baseline (speedup 1.0000x reference)
import functools

import jax
import jax.numpy as jnp
from jax import lax
from jax.experimental import pallas as pl
from jax.experimental.pallas import tpu as pltpu
from jax.experimental.pallas import tpu_sc as plsc

VOCAB = 100000
EMBED = 64
B = 1024
CTX = 20

NC = 2
NS = 16
NW = NC * NS
B_PER_W = B // NW
IDX_PER_W = B_PER_W * CTX
IDX_CHUNK = 128
N_CHUNKS = IDX_PER_W // IDX_CHUNK

LANES = 16
EC = EMBED // LANES

V_TILE = 1024


def _gather_mean_sc(table, idx3):
    mesh = plsc.VectorSubcoreMesh(core_axis_name="c", subcore_axis_name="s")

    @functools.partial(
        pl.kernel,
        mesh=mesh,
        compiler_params=pltpu.CompilerParams(use_tc_tiling_on_sc=False),
        out_type=jax.ShapeDtypeStruct((B, EMBED), jnp.float32),
        scratch_types=[
            pltpu.VMEM((N_CHUNKS, IDX_CHUNK), jnp.int32),
            pltpu.VMEM((IDX_PER_W, EMBED), jnp.float32),
            pltpu.VMEM((B_PER_W, EMBED), jnp.float32),
            pltpu.SemaphoreType.DMA,
        ],
    )
    def k(table_hbm, idx_hbm, out_hbm, idx_v, rows_v, acc_v, sem):
        wid = lax.axis_index("s") * NC + lax.axis_index("c")
        pltpu.sync_copy(idx_hbm.at[wid], idx_v)
        copies = [
            pltpu.async_copy(
                table_hbm.at[idx_v.at[j]],
                rows_v.at[pl.ds(j * IDX_CHUNK, IDX_CHUNK)],
                sem,
            )
            for j in range(N_CHUNKS)
        ]
        for c in copies:
            c.wait()

        def body(b, carry):
            for c in range(EC):
                acc = jnp.zeros((LANES,), jnp.float32)
                for t in range(CTX):
                    acc = acc + rows_v[b * CTX + t, pl.ds(c * LANES, LANES)]
                acc_v[b, pl.ds(c * LANES, LANES)] = acc * (1.0 / CTX)
            return carry

        lax.fori_loop(0, B_PER_W, body, 0)
        pltpu.sync_copy(acc_v, out_hbm.at[pl.ds(wid * B_PER_W, B_PER_W)])

    return k(table, idx3)


def _project_tc(avg, lin_w, lin_b2):
    def mm(avg_ref, w_ref, b_ref, out_ref):
        out_ref[...] = lax.dot_general(
            avg_ref[...], w_ref[...],
            (((1,), (1,)), ((), ())),
            preferred_element_type=jnp.float32,
        ) + b_ref[...]

    return pl.pallas_call(
        mm,
        grid=(pl.cdiv(VOCAB, V_TILE),),
        in_specs=[
            pl.BlockSpec((B, EMBED), lambda j: (0, 0)),
            pl.BlockSpec((V_TILE, EMBED), lambda j: (j, 0)),
            pl.BlockSpec((1, V_TILE), lambda j: (0, j)),
        ],
        out_specs=pl.BlockSpec((B, V_TILE), lambda j: (0, j)),
        out_shape=jax.ShapeDtypeStruct((B, VOCAB), jnp.float32),
    )(avg, lin_w, lin_b2)


def kernel(inputs, emb_table, lin_w, lin_b):
    idx3 = inputs.astype(jnp.int32).reshape(NW, N_CHUNKS, IDX_CHUNK)
    avg = _gather_mean_sc(emb_table, idx3)
    return _project_tc(avg, lin_w, lin_b.reshape(1, VOCAB))
